# Initial kernel scaffold; baseline (speedup 1.0000x reference)
import jax
import jax.numpy as jnp
from jax.experimental import pallas as pl


def kernel(vertices, faces):
    raise NotImplementedError("write your pallas kernel here")


# speedup vs baseline: 25.7097x; 25.7097x over previous
import functools

import jax
import jax.numpy as jnp
from jax import lax
from jax.experimental import pallas as pl
from jax.experimental.pallas import tpu as pltpu
from jax.experimental.pallas import tpu_sc as plsc

V = 50000
N = 4
VP = 50176
SLICE = VP // 16
E = 3 * 100000
EP = 307200
TILE_E = EP // 16
CHUNK = 128
NCHUNK = TILE_E // CHUNK


def _edge_chunk(keys_v, prev_v, base, j):
    kv = keys_v[pl.ds(base + j * 16, 16)]
    pv = prev_v[pl.ds(base + j * 16, 16)]
    m = kv != pv
    shift = jnp.full((16,), 16, dtype=jnp.int32)
    e0 = lax.shift_right_logical(kv, shift)
    e1 = jnp.bitwise_and(kv, jnp.int32(0xFFFF))
    return m, e0, e1


def _sc_body(keys_h, prev_h, vt_h, s_h, deg_h,
             plane_v, keys_v, prev_v, zero_v, idx0_v, idx1_v, val0_v, val1_v,
             out_v, acc_sh):
    c = lax.axis_index("c")
    s = lax.axis_index("s")
    ebase = s * TILE_E

    pltpu.sync_copy(keys_h.at[pl.ds(ebase, TILE_E)], keys_v)
    pltpu.sync_copy(prev_h.at[pl.ds(ebase, TILE_E)], prev_v)

    def _zbody(i, carry):
        zero_v[pl.ds(i * 16, 16)] = jnp.zeros((16,), jnp.float32)
        return carry
    lax.fori_loop(0, SLICE // 16, _zbody, 0)

    def _scatter_pass(gather_plane):
        def _chunk(k, carry):
            base = k * CHUNK
            for j in range(CHUNK // 16):
                m, e0, e1 = _edge_chunk(keys_v, prev_v, base, j)
                zero16 = jnp.zeros((16,), jnp.float32)
                if gather_plane:
                    g1 = plsc.load_gather(plane_v, [e1])
                    g0 = plsc.load_gather(plane_v, [e0])
                    v0 = jnp.where(m, g1, zero16)
                    v1 = jnp.where(m, g0, zero16)
                else:
                    mv = jnp.where(m, jnp.full((16,), 1.0, jnp.float32), zero16)
                    v0 = mv
                    v1 = mv
                idx0_v[pl.ds(j * 16, 16)] = e0
                val0_v[pl.ds(j * 16, 16)] = v0
                idx1_v[pl.ds(j * 16, 16)] = e1
                val1_v[pl.ds(j * 16, 16)] = v1
            pltpu.sync_copy(val0_v, acc_sh.at[idx0_v], add=True)
            pltpu.sync_copy(val1_v, acc_sh.at[idx1_v], add=True)
            return carry
        lax.fori_loop(0, NCHUNK, _chunk, 0)

    for p in range(6):
        plane_idx = 6 * c + p
        pltpu.sync_copy(vt_h.at[pl.ds(plane_idx * VP, VP)], plane_v)
        pltpu.sync_copy(zero_v, acc_sh.at[pl.ds(s * SLICE, SLICE)])
        plsc.subcore_barrier()
        _scatter_pass(gather_plane=True)
        plsc.subcore_barrier()
        pltpu.sync_copy(acc_sh.at[pl.ds(s * SLICE, SLICE)], out_v)
        pltpu.sync_copy(out_v, s_h.at[pl.ds(plane_idx * VP + s * SLICE, SLICE)])

    @pl.when(c == 0)
    def _deg_pass():
        pltpu.sync_copy(zero_v, acc_sh.at[pl.ds(s * SLICE, SLICE)])
        plsc.subcore_barrier()
        _scatter_pass(gather_plane=False)
        plsc.subcore_barrier()
        pltpu.sync_copy(acc_sh.at[pl.ds(s * SLICE, SLICE)], out_v)
        pltpu.sync_copy(out_v, deg_h.at[pl.ds(s * SLICE, SLICE)])


_sc_kernel = functools.partial(
    pl.kernel,
    out_type=(
        jax.ShapeDtypeStruct((N * 3 * VP,), jnp.float32),
        jax.ShapeDtypeStruct((VP,), jnp.float32),
    ),
    mesh=plsc.VectorSubcoreMesh(core_axis_name="c", subcore_axis_name="s"),
    scratch_types=(
        pltpu.VMEM((VP,), jnp.float32),
        pltpu.VMEM((TILE_E,), jnp.int32),
        pltpu.VMEM((TILE_E,), jnp.int32),
        pltpu.VMEM((SLICE,), jnp.float32),
        pltpu.VMEM((CHUNK,), jnp.int32),
        pltpu.VMEM((CHUNK,), jnp.int32),
        pltpu.VMEM((CHUNK,), jnp.float32),
        pltpu.VMEM((CHUNK,), jnp.float32),
        pltpu.VMEM((SLICE,), jnp.float32),
        pltpu.VMEM_SHARED((VP,), jnp.float32),
    ),
    compiler_params=pltpu.CompilerParams(needs_layout_passes=False),
)(_sc_body)


def _tc_finish(s_ref, deg_ref, vt_ref, out_ref):
    S = s_ref[...]
    vt = vt_ref[...]
    d = deg_ref[...]
    inv = jnp.where(d > 0.0, 1.0 / jnp.where(d > 0.0, d, 1.0), 0.0)
    R = S * inv - vt
    sq = jnp.sum(R * R, axis=1)
    out_ref[0, 0] = jnp.sum(jnp.sqrt(sq)) * (1.0 / (V * N))


def kernel(vertices, faces):
    f = faces.astype(jnp.int32)
    x = jnp.concatenate([f[:, 0], f[:, 1], f[:, 2]])
    y = jnp.concatenate([f[:, 1], f[:, 2], f[:, 0]])
    a = jnp.minimum(x, y).astype(jnp.uint32)
    b = jnp.maximum(x, y).astype(jnp.uint32)
    keys = (a << 16) | b
    sk = jnp.sort(keys)
    prev = jnp.concatenate([sk[:1] ^ jnp.uint32(1), sk[:-1]])
    pad_i = jnp.arange(EP - E, dtype=jnp.uint32)
    padk = (pad_i << 16) | pad_i
    keys_full = lax.bitcast_convert_type(jnp.concatenate([sk, padk]), jnp.int32)
    prev_full = lax.bitcast_convert_type(jnp.concatenate([prev, padk]), jnp.int32)

    vt = jnp.pad(jnp.transpose(vertices, (0, 2, 1)),
                 ((0, 0), (0, 0), (0, VP - V)))
    vt_flat = vt.reshape(N * 3 * VP)

    S_flat, deg = _sc_kernel(keys_full, prev_full, vt_flat)

    total = pl.pallas_call(
        _tc_finish,
        out_shape=jax.ShapeDtypeStruct((1, 1), jnp.float32),
        out_specs=pl.BlockSpec(memory_space=pltpu.SMEM),
    )(S_flat.reshape(N, 3, VP), deg.reshape(1, 1, VP), vt)
    return total[0, 0]


# speedup vs baseline: 33.1297x; 1.2886x over previous
import functools

import jax
import jax.numpy as jnp
from jax import lax
from jax.experimental import pallas as pl
from jax.experimental.pallas import tpu as pltpu
from jax.experimental.pallas import tpu_sc as plsc

V = 50000
N = 4
VP = 50176
SLICE = VP // 16
E = 3 * 100000
EP = 307200
TILE_E = EP // 16
CHUNK = 128
NCHUNK = TILE_E // CHUNK


def _edge_chunk(keys_v, prev_v, base, j):
    kv = keys_v[pl.ds(base + j * 16, 16)]
    pv = prev_v[pl.ds(base + j * 16, 16)]
    m = kv != pv
    shift = jnp.full((16,), 16, dtype=jnp.int32)
    e0 = lax.shift_right_logical(kv, shift)
    e1 = jnp.bitwise_and(kv, jnp.int32(0xFFFF))
    return m, e0, e1


def _sc_body(keys_h, prev_h, vt_h, s_h, deg_h,
             plane_v, keys_v, prev_v, zero_v,
             idx0_a, idx1_a, val0_a, val1_a,
             idx0_b, idx1_b, val0_b, val1_b,
             out_v, acc_sh, sem_a, sem_b):
    c = lax.axis_index("c")
    s = lax.axis_index("s")
    ebase = s * TILE_E

    pltpu.sync_copy(keys_h.at[pl.ds(ebase, TILE_E)], keys_v)
    pltpu.sync_copy(prev_h.at[pl.ds(ebase, TILE_E)], prev_v)

    def _zbody(i, carry):
        zero_v[pl.ds(i * 16, 16)] = jnp.zeros((16,), jnp.float32)
        return carry
    lax.fori_loop(0, SLICE // 16, _zbody, 0)

    buf_a = (idx0_a, idx1_a, val0_a, val1_a, sem_a)
    buf_b = (idx0_b, idx1_b, val0_b, val1_b, sem_b)

    def _compute_chunk(k, buf, gather_plane):
        idx0_v, idx1_v, val0_v, val1_v, _ = buf
        base = k * CHUNK
        for j in range(CHUNK // 16):
            m, e0, e1 = _edge_chunk(keys_v, prev_v, base, j)
            zero16 = jnp.zeros((16,), jnp.float32)
            if gather_plane:
                g1 = plsc.load_gather(plane_v, [e1])
                g0 = plsc.load_gather(plane_v, [e0])
                v0 = jnp.where(m, g1, zero16)
                v1 = jnp.where(m, g0, zero16)
            else:
                mv = jnp.where(m, jnp.full((16,), 1.0, jnp.float32), zero16)
                v0 = mv
                v1 = mv
            idx0_v[pl.ds(j * 16, 16)] = e0
            val0_v[pl.ds(j * 16, 16)] = v0
            idx1_v[pl.ds(j * 16, 16)] = e1
            val1_v[pl.ds(j * 16, 16)] = v1

    def _fire(buf):
        idx0_v, idx1_v, val0_v, val1_v, sem = buf
        pltpu.async_copy(val0_v, acc_sh.at[idx0_v], sem, add=True)
        pltpu.async_copy(val1_v, acc_sh.at[idx1_v], sem, add=True)

    def _drain(buf):
        idx0_v, idx1_v, val0_v, val1_v, sem = buf
        pltpu.make_async_copy(val0_v, acc_sh.at[idx0_v], sem).wait()
        pltpu.make_async_copy(val1_v, acc_sh.at[idx1_v], sem).wait()

    def _scatter_pass(gather_plane):
        _compute_chunk(0, buf_a, gather_plane)
        _fire(buf_a)
        _compute_chunk(1, buf_b, gather_plane)
        _fire(buf_b)

        def _pair(i, carry):
            _drain(buf_a)
            _compute_chunk(2 * i + 2, buf_a, gather_plane)
            _fire(buf_a)
            _drain(buf_b)
            _compute_chunk(2 * i + 3, buf_b, gather_plane)
            _fire(buf_b)
            return carry
        lax.fori_loop(0, (NCHUNK - 2) // 2, _pair, 0)
        _drain(buf_a)
        _drain(buf_b)

    for p in range(6):
        plane_idx = 6 * c + p
        pltpu.sync_copy(vt_h.at[pl.ds(plane_idx * VP, VP)], plane_v)
        pltpu.sync_copy(zero_v, acc_sh.at[pl.ds(s * SLICE, SLICE)])
        plsc.subcore_barrier()
        _scatter_pass(gather_plane=True)
        plsc.subcore_barrier()
        pltpu.sync_copy(acc_sh.at[pl.ds(s * SLICE, SLICE)], out_v)
        pltpu.sync_copy(out_v, s_h.at[pl.ds(plane_idx * VP + s * SLICE, SLICE)])

    @pl.when(c == 0)
    def _deg_pass():
        pltpu.sync_copy(zero_v, acc_sh.at[pl.ds(s * SLICE, SLICE)])
        plsc.subcore_barrier()
        _scatter_pass(gather_plane=False)
        plsc.subcore_barrier()
        pltpu.sync_copy(acc_sh.at[pl.ds(s * SLICE, SLICE)], out_v)
        pltpu.sync_copy(out_v, deg_h.at[pl.ds(s * SLICE, SLICE)])


_sc_kernel = functools.partial(
    pl.kernel,
    out_type=(
        jax.ShapeDtypeStruct((N * 3 * VP,), jnp.float32),
        jax.ShapeDtypeStruct((VP,), jnp.float32),
    ),
    mesh=plsc.VectorSubcoreMesh(core_axis_name="c", subcore_axis_name="s"),
    scratch_types=(
        pltpu.VMEM((VP,), jnp.float32),
        pltpu.VMEM((TILE_E,), jnp.int32),
        pltpu.VMEM((TILE_E,), jnp.int32),
        pltpu.VMEM((SLICE,), jnp.float32),
        pltpu.VMEM((CHUNK,), jnp.int32),
        pltpu.VMEM((CHUNK,), jnp.int32),
        pltpu.VMEM((CHUNK,), jnp.float32),
        pltpu.VMEM((CHUNK,), jnp.float32),
        pltpu.VMEM((CHUNK,), jnp.int32),
        pltpu.VMEM((CHUNK,), jnp.int32),
        pltpu.VMEM((CHUNK,), jnp.float32),
        pltpu.VMEM((CHUNK,), jnp.float32),
        pltpu.VMEM((SLICE,), jnp.float32),
        pltpu.VMEM_SHARED((VP,), jnp.float32),
        pltpu.SemaphoreType.DMA,
        pltpu.SemaphoreType.DMA,
    ),
    compiler_params=pltpu.CompilerParams(needs_layout_passes=False),
)(_sc_body)


def _tc_finish(s_ref, deg_ref, vt_ref, out_ref):
    S = s_ref[...]
    vt = vt_ref[...]
    d = deg_ref[...]
    inv = jnp.where(d > 0.0, 1.0 / jnp.where(d > 0.0, d, 1.0), 0.0)
    R = S * inv - vt
    sq = jnp.sum(R * R, axis=1)
    out_ref[0, 0] = jnp.sum(jnp.sqrt(sq)) * (1.0 / (V * N))


def kernel(vertices, faces):
    f = faces.astype(jnp.int32)
    x = jnp.concatenate([f[:, 0], f[:, 1], f[:, 2]])
    y = jnp.concatenate([f[:, 1], f[:, 2], f[:, 0]])
    a = jnp.minimum(x, y).astype(jnp.uint32)
    b = jnp.maximum(x, y).astype(jnp.uint32)
    keys = (a << 16) | b
    sk = jnp.sort(keys)
    prev = jnp.concatenate([sk[:1] ^ jnp.uint32(1), sk[:-1]])
    pad_i = jnp.arange(EP - E, dtype=jnp.uint32)
    padk = (pad_i << 16) | pad_i
    keys_full = lax.bitcast_convert_type(jnp.concatenate([sk, padk]), jnp.int32)
    prev_full = lax.bitcast_convert_type(jnp.concatenate([prev, padk]), jnp.int32)

    vt = jnp.pad(jnp.transpose(vertices, (0, 2, 1)),
                 ((0, 0), (0, 0), (0, VP - V)))
    vt_flat = vt.reshape(N * 3 * VP)

    S_flat, deg = _sc_kernel(keys_full, prev_full, vt_flat)

    total = pl.pallas_call(
        _tc_finish,
        out_shape=jax.ShapeDtypeStruct((1, 1), jnp.float32),
        out_specs=pl.BlockSpec(memory_space=pltpu.SMEM),
    )(S_flat.reshape(N, 3, VP), deg.reshape(1, 1, VP), vt)
    return total[0, 0]
